# Initial kernel scaffold; baseline (speedup 1.0000x reference)
import jax
import jax.numpy as jnp
from jax.experimental import pallas as pl


def kernel(x, weight):
    raise NotImplementedError("write your pallas kernel here")


# speedup vs baseline: 2.9957x; 2.9957x over previous
import functools

import jax
import jax.numpy as jnp
from jax import lax
from jax.experimental import pallas as pl
from jax.experimental.pallas import tpu as pltpu
from jax.experimental.pallas import tpu_sc as plsc

NUM_CORES = 2
NUM_SUBCORES = 16
NUM_WORKERS = NUM_CORES * NUM_SUBCORES

D = 128
CHUNK = 128


def _gather_body(b_per_w, n_chunks, x_hbm, w_hbm, out_hbm, idx_v, rows_v, sem):
  wid = lax.axis_index("s") * NUM_CORES + lax.axis_index("c")
  base = wid * b_per_w

  pltpu.sync_copy(x_hbm.at[pl.ds(base, b_per_w)], idx_v)

  def chunk_body(c, carry):
    off = c * CHUNK
    cp = pltpu.async_copy(w_hbm.at[idx_v.at[pl.ds(off, CHUNK)]], rows_v, sem)
    cp.wait()
    pltpu.sync_copy(rows_v, out_hbm.at[pl.ds(base + off, CHUNK)])
    return carry

  lax.fori_loop(0, n_chunks, chunk_body, 0, unroll=False)


def kernel(x, weight):
  B = x.shape[0] * x.shape[1]
  assert B % (NUM_WORKERS * CHUNK) == 0
  b_per_w = B // NUM_WORKERS
  n_chunks = b_per_w // CHUNK

  flat_x = x.reshape(B)

  mesh = plsc.VectorSubcoreMesh(
      core_axis_name="c", subcore_axis_name="s",
      num_cores=NUM_CORES, num_subcores=NUM_SUBCORES)

  grid_kernel = pl.kernel(
      functools.partial(_gather_body, b_per_w, n_chunks),
      out_type=jax.ShapeDtypeStruct((B, D), jnp.float32),
      mesh=mesh,
      scratch_types=[
          pltpu.VMEM((b_per_w,), jnp.int32),
          pltpu.VMEM((CHUNK, D), jnp.float32),
          pltpu.SemaphoreType.DMA,
      ],
  )
  out = grid_kernel(flat_x, weight)
  return out.reshape(x.shape[0], x.shape[1], D)


# speedup vs baseline: 3.3667x; 1.1238x over previous
import functools

import jax
import jax.numpy as jnp
from jax import lax
from jax.experimental import pallas as pl
from jax.experimental.pallas import tpu as pltpu
from jax.experimental.pallas import tpu_sc as plsc

NUM_CORES = 2
NUM_SUBCORES = 16
NUM_WORKERS = NUM_CORES * NUM_SUBCORES

D = 128
CHUNK = 128


def _gather_body(n_chunks, x_hbm, w_hbm, out_hbm,
                 idx_v, buf0, buf1, gsem0, gsem1, wsem0, wsem1):
  wid = lax.axis_index("s") * NUM_CORES + lax.axis_index("c")
  base = wid * n_chunks

  pltpu.sync_copy(x_hbm.at[pl.ds(base, n_chunks)], idx_v)

  def start_gather(c, buf, sem):
    pltpu.async_copy(w_hbm.at[idx_v.at[c]], buf, sem)

  def wait_gather(buf, sem):
    pltpu.make_async_copy(w_hbm.at[idx_v.at[0]], buf, sem).wait()

  def start_write(c, buf, sem):
    pltpu.async_copy(buf, out_hbm.at[pl.ds((base + c) * CHUNK, CHUNK)], sem)

  def wait_write(buf, sem):
    pltpu.make_async_copy(buf, out_hbm.at[pl.ds(0, CHUNK)], sem).wait()

  start_gather(0, buf0, gsem0)
  start_gather(1, buf1, gsem1)
  wait_gather(buf0, gsem0)
  start_write(0, buf0, wsem0)

  def loop_body(j, carry):
    wait_write(buf0, wsem0)
    start_gather(2 * j, buf0, gsem0)
    wait_gather(buf1, gsem1)
    start_write(2 * j - 1, buf1, wsem1)
    wait_write(buf1, wsem1)
    start_gather(2 * j + 1, buf1, gsem1)
    wait_gather(buf0, gsem0)
    start_write(2 * j, buf0, wsem0)
    return carry

  lax.fori_loop(1, n_chunks // 2, loop_body, 0, unroll=False)

  wait_gather(buf1, gsem1)
  start_write(n_chunks - 1, buf1, wsem1)
  wait_write(buf0, wsem0)
  wait_write(buf1, wsem1)


def kernel(x, weight):
  B = x.shape[0] * x.shape[1]
  assert B % (NUM_WORKERS * CHUNK) == 0
  b_per_w = B // NUM_WORKERS
  n_chunks = b_per_w // CHUNK
  assert n_chunks % 2 == 0

  flat_x = x.reshape(B // CHUNK, CHUNK)

  mesh = plsc.VectorSubcoreMesh(
      core_axis_name="c", subcore_axis_name="s",
      num_cores=NUM_CORES, num_subcores=NUM_SUBCORES)

  grid_kernel = pl.kernel(
      functools.partial(_gather_body, n_chunks),
      out_type=jax.ShapeDtypeStruct((B, D), jnp.float32),
      mesh=mesh,
      scratch_types=[
          pltpu.VMEM((n_chunks, CHUNK), jnp.int32),
          pltpu.VMEM((CHUNK, D), jnp.float32),
          pltpu.VMEM((CHUNK, D), jnp.float32),
          pltpu.SemaphoreType.DMA,
          pltpu.SemaphoreType.DMA,
          pltpu.SemaphoreType.DMA,
          pltpu.SemaphoreType.DMA,
      ],
  )
  out = grid_kernel(flat_x, weight)
  return out.reshape(x.shape[0], x.shape[1], D)


# speedup vs baseline: 3.3851x; 1.0055x over previous
import functools

import jax
import jax.numpy as jnp
from jax import lax
from jax.experimental import pallas as pl
from jax.experimental.pallas import tpu as pltpu
from jax.experimental.pallas import tpu_sc as plsc

NUM_CORES = 2
NUM_SUBCORES = 16
NUM_WORKERS = NUM_CORES * NUM_SUBCORES

D = 128
CHUNK = 128


NBUF = 4


def _gather_body(n_chunks, x_hbm, w_hbm, out_hbm, idx_v, bufs, gsems, wsems):
  wid = lax.axis_index("s") * NUM_CORES + lax.axis_index("c")
  base = wid * n_chunks

  pltpu.sync_copy(x_hbm.at[pl.ds(base, n_chunks)], idx_v)

  def start_gather(c, b):
    pltpu.async_copy(w_hbm.at[idx_v.at[c]], bufs[b], gsems[b])

  def wait_gather(b):
    pltpu.make_async_copy(w_hbm.at[idx_v.at[0]], bufs[b], gsems[b]).wait()

  def start_write(c, b):
    pltpu.async_copy(bufs[b], out_hbm.at[pl.ds((base + c) * CHUNK, CHUNK)],
                     wsems[b])

  def wait_write(b):
    pltpu.make_async_copy(bufs[b], out_hbm.at[pl.ds(0, CHUNK)], wsems[b]).wait()

  for b in range(NBUF):
    start_gather(b, b)

  n_groups = n_chunks // NBUF

  def loop_body(j, carry):
    for b in range(NBUF):
      c = NBUF * j + b
      wait_gather(b)
      start_write(c, b)
      wait_write(b)
      start_gather(c + NBUF, b)
    return carry

  lax.fori_loop(0, n_groups - 1, loop_body, 0, unroll=False)

  for b in range(NBUF):
    c = NBUF * (n_groups - 1) + b
    wait_gather(b)
    start_write(c, b)
  for b in range(NBUF):
    wait_write(b)


def kernel(x, weight):
  B = x.shape[0] * x.shape[1]
  assert B % (NUM_WORKERS * CHUNK) == 0
  b_per_w = B // NUM_WORKERS
  n_chunks = b_per_w // CHUNK
  assert n_chunks % NBUF == 0

  flat_x = x.reshape(B // CHUNK, CHUNK)

  mesh = plsc.VectorSubcoreMesh(
      core_axis_name="c", subcore_axis_name="s",
      num_cores=NUM_CORES, num_subcores=NUM_SUBCORES)

  grid_kernel = pl.kernel(
      functools.partial(_gather_body, n_chunks),
      out_type=jax.ShapeDtypeStruct((B, D), jnp.float32),
      mesh=mesh,
      scratch_types=[
          pltpu.VMEM((n_chunks, CHUNK), jnp.int32),
          [pltpu.VMEM((CHUNK, D), jnp.float32) for _ in range(NBUF)],
          [pltpu.SemaphoreType.DMA for _ in range(NBUF)],
          [pltpu.SemaphoreType.DMA for _ in range(NBUF)],
      ],
  )
  out = grid_kernel(flat_x, weight)
  return out.reshape(x.shape[0], x.shape[1], D)


# speedup vs baseline: 5.6654x; 1.6736x over previous
import functools

import jax
import jax.numpy as jnp
from jax import lax
from jax.experimental import pallas as pl
from jax.experimental.pallas import tpu as pltpu
from jax.experimental.pallas import tpu_sc as plsc

NUM_CORES = 2
NUM_SUBCORES = 16
NUM_WORKERS = NUM_CORES * NUM_SUBCORES

D = 128
SEG = 26
SLABS_PER_CHUNK = 4
CHUNK = SLABS_PER_CHUNK * SEG
NBUF = 4


def _gather_body(n_chunks, x_hbm, w_hbm, out_hbm, idx_v, bufs, gsems, wsems):
  wid = lax.axis_index("s") * NUM_CORES + lax.axis_index("c")
  b_per_w = n_chunks * CHUNK
  base_slab = wid * (n_chunks * SLABS_PER_CHUNK)

  pltpu.sync_copy(x_hbm.at[pl.ds(wid * b_per_w, b_per_w)], idx_v)

  def start_gather(c, b):
    pltpu.async_copy(w_hbm.at[idx_v.at[pl.ds(c * CHUNK, CHUNK)]],
                     bufs[b], gsems[b])

  def wait_gather(b):
    pltpu.make_async_copy(w_hbm.at[idx_v.at[pl.ds(0, CHUNK)]],
                          bufs[b], gsems[b]).wait()

  def start_write(c, b):
    for k in range(SLABS_PER_CHUNK):
      pltpu.async_copy(bufs[b].at[pl.ds(k * SEG, SEG)],
                       out_hbm.at[base_slab + c * SLABS_PER_CHUNK + k],
                       wsems[b])

  def wait_write(b):
    for _ in range(SLABS_PER_CHUNK):
      pltpu.make_async_copy(bufs[b].at[pl.ds(0, SEG)], out_hbm.at[0],
                            wsems[b]).wait()

  for b in range(NBUF):
    start_gather(b, b)

  n_groups = n_chunks // NBUF

  def loop_body(j, carry):
    for b in range(NBUF):
      c = NBUF * j + b
      wait_gather(b)
      start_write(c, b)
      wait_write(b)
      start_gather(c + NBUF, b)
    return carry

  lax.fori_loop(0, n_groups - 1, loop_body, 0, unroll=False)

  for b in range(NBUF):
    c = NBUF * (n_groups - 1) + b
    wait_gather(b)
    start_write(c, b)
  for b in range(NBUF):
    wait_write(b)


def kernel(x, weight):
  N, S = x.shape
  B = N * S
  assert S == SEG and N % (NUM_WORKERS * SLABS_PER_CHUNK) == 0
  slabs_per_w = N // NUM_WORKERS
  n_chunks = slabs_per_w // SLABS_PER_CHUNK
  assert n_chunks % NBUF == 0

  flat_x = x.reshape(B)

  mesh = plsc.VectorSubcoreMesh(
      core_axis_name="c", subcore_axis_name="s",
      num_cores=NUM_CORES, num_subcores=NUM_SUBCORES)

  grid_kernel = pl.kernel(
      functools.partial(_gather_body, n_chunks),
      out_type=jax.ShapeDtypeStruct((N, S, D), jnp.float32),
      mesh=mesh,
      compiler_params=pltpu.CompilerParams(use_tc_tiling_on_sc=True),
      scratch_types=[
          pltpu.VMEM((n_chunks * CHUNK,), jnp.int32),
          [pltpu.VMEM((CHUNK, D), jnp.float32) for _ in range(NBUF)],
          [pltpu.SemaphoreType.DMA for _ in range(NBUF)],
          [pltpu.SemaphoreType.DMA for _ in range(NBUF)],
      ],
  )
  return grid_kernel(flat_x, weight)


# speedup vs baseline: 11.9420x; 2.1079x over previous
import functools

import jax
import jax.numpy as jnp
from jax import lax
from jax.experimental import pallas as pl
from jax.experimental.pallas import tpu as pltpu
from jax.experimental.pallas import tpu_sc as plsc

NUM_CORES = 2
NUM_SUBCORES = 16
NUM_WORKERS = NUM_CORES * NUM_SUBCORES

D = 128
GATHER = 128
GPC = 2
CHUNK = GPC * GATHER
NBUF = 2


def _gather_body(n_chunks, x_hbm, w_hbm, out_hbm, idx_v, bufs, gsems, wsems):
  wid = lax.axis_index("s") * NUM_CORES + lax.axis_index("c")
  base = wid * n_chunks * GPC

  pltpu.sync_copy(x_hbm.at[pl.ds(base, n_chunks * GPC)], idx_v)

  def start_gather(c, b):
    for g in range(GPC):
      pltpu.async_copy(w_hbm.at[idx_v.at[c * GPC + g]],
                       bufs[b].at[pl.ds(g * GATHER, GATHER)], gsems[b])

  def wait_gather(b):
    for _ in range(GPC):
      pltpu.make_async_copy(w_hbm.at[idx_v.at[0]],
                            bufs[b].at[pl.ds(0, GATHER)], gsems[b]).wait()

  def start_write(c, b):
    pltpu.async_copy(bufs[b], out_hbm.at[pl.ds((base + c * GPC) * GATHER,
                                               CHUNK)], wsems[b])

  def wait_write(b):
    pltpu.make_async_copy(bufs[b], out_hbm.at[pl.ds(0, CHUNK)], wsems[b]).wait()

  for b in range(NBUF):
    start_gather(b, b)

  n_groups = n_chunks // NBUF

  def loop_body(j, carry):
    for b in range(NBUF):
      c = NBUF * j + b
      wait_gather(b)
      start_write(c, b)
      wait_write(b)
      start_gather(c + NBUF, b)
    return carry

  lax.fori_loop(0, n_groups - 1, loop_body, 0, unroll=False)

  for b in range(NBUF):
    c = NBUF * (n_groups - 1) + b
    wait_gather(b)
    start_write(c, b)
  for b in range(NBUF):
    wait_write(b)


def kernel(x, weight):
  N, S = x.shape
  B = N * S
  assert B % (NUM_WORKERS * CHUNK) == 0
  b_per_w = B // NUM_WORKERS
  n_chunks = b_per_w // CHUNK
  assert n_chunks % NBUF == 0

  flat_x = jnp.transpose(x, (1, 0)).reshape(B // GATHER, GATHER)

  mesh = plsc.VectorSubcoreMesh(
      core_axis_name="c", subcore_axis_name="s",
      num_cores=NUM_CORES, num_subcores=NUM_SUBCORES)

  grid_kernel = pl.kernel(
      functools.partial(_gather_body, n_chunks),
      out_type=jax.ShapeDtypeStruct((B, D), jnp.float32),
      mesh=mesh,
      scratch_types=[
          pltpu.VMEM((n_chunks * GPC, GATHER), jnp.int32),
          [pltpu.VMEM((CHUNK, D), jnp.float32) for _ in range(NBUF)],
          [pltpu.SemaphoreType.DMA for _ in range(NBUF)],
          [pltpu.SemaphoreType.DMA for _ in range(NBUF)],
      ],
  )
  out = grid_kernel(flat_x, weight)
  return out.reshape(S, N, D).transpose(1, 0, 2)
